# Initial kernel scaffold; baseline (speedup 1.0000x reference)
import jax
import jax.numpy as jnp
from jax.experimental import pallas as pl


def kernel(inputs, embeddings):
    raise NotImplementedError("write your pallas kernel here")


# speedup vs baseline: 1.0258x; 1.0258x over previous
import functools

import jax
import jax.numpy as jnp
from jax.experimental import pallas as pl
from jax.experimental.pallas import tpu as pltpu
from jax import lax

B = 256
FEAT = 32
BOX = 8
K = 1024
EMB_DIM = BOX * BOX * BOX * FEAT

KB = 128


def _vq_body(x_ref, e_ref, out_ref, minv_ref, mini_ref):
    j = pl.program_id(0)
    x = x_ref[...]
    e = e_ref[...]
    mm = lax.dot_general(
        x, e, (((1,), (1,)), ((), ())), preferred_element_type=jnp.float32
    )
    x_sq = jnp.sum(x * x, axis=1, keepdims=True)
    e_sq = jnp.sum(e * e, axis=1)
    dist = (x_sq + e_sq[None, :]) - 2.0 * mm

    local_min = jnp.min(dist, axis=1, keepdims=True)
    iota = lax.broadcasted_iota(jnp.int32, dist.shape, 1) + j * KB
    local_arg = jnp.min(
        jnp.where(dist <= local_min, iota, K), axis=1, keepdims=True
    )

    @pl.when(j == 0)
    def _init():
        minv_ref[...] = local_min
        mini_ref[...] = local_arg

    @pl.when(j > 0)
    def _merge():
        better = local_min < minv_ref[...]
        minv_ref[...] = jnp.where(better, local_min, minv_ref[...])
        mini_ref[...] = jnp.where(better, local_arg, mini_ref[...])

    @pl.when(j == pl.num_programs(0) - 1)
    def _finish():
        out_ref[...] = mini_ref[...]


@functools.partial(jax.jit, static_argnames=())
def kernel(inputs, embeddings):
    x = inputs.reshape(B, EMB_DIM)
    grid = (K // KB,)
    out = pl.pallas_call(
        _vq_body,
        grid=grid,
        in_specs=[
            pl.BlockSpec((B, EMB_DIM), lambda j: (0, 0)),
            pl.BlockSpec((KB, EMB_DIM), lambda j: (j, 0)),
        ],
        out_specs=pl.BlockSpec((B, 1), lambda j: (0, 0)),
        out_shape=jax.ShapeDtypeStruct((B, 1), jnp.int32),
        scratch_shapes=[
            pltpu.VMEM((B, 1), jnp.float32),
            pltpu.VMEM((B, 1), jnp.int32),
        ],
    )(x, embeddings)
    return out.reshape(B)
